# Initial kernel scaffold; baseline (speedup 1.0000x reference)
import jax
import jax.numpy as jnp
from jax.experimental import pallas as pl


def kernel(concept_hidden, head, tail, W, scoring_fn_source, scoring_fn_target):
    raise NotImplementedError("write your pallas kernel here")


# speedup vs baseline: 4.3639x; 4.3639x over previous
import functools

import jax
import jax.numpy as jnp
from jax import lax
from jax.experimental import pallas as pl
from jax.experimental.pallas import tpu as pltpu
from jax.experimental.pallas import tpu_sc as plsc

N = 10000
E = 320000
IN = 128
H = 8
OUT = 16

NC = 2
NS = 16
NW = NC * NS
PER_W = E // NW
SUB = 125
ROWS_PER_CHUNK = 16
C = SUB * ROWS_PER_CHUNK
NCHUNK = PER_W // C


def _tc_tables(x_ref, w_ref, sa_ref, sb_ref, a_ref, b_ref):
    wa = jnp.dot(w_ref[...], sa_ref[...], preferred_element_type=jnp.float32)
    wb = jnp.dot(w_ref[...], sb_ref[...], preferred_element_type=jnp.float32)
    x = x_ref[...]
    a_ref[...] = jnp.dot(x, wa, preferred_element_type=jnp.float32)
    b_ref[...] = jnp.dot(x, wb, preferred_element_type=jnp.float32)


def _make_tables(x, W, S_a, S_b):
    return pl.pallas_call(
        _tc_tables,
        out_shape=(
            jax.ShapeDtypeStruct((N, 16), jnp.float32),
            jax.ShapeDtypeStruct((N, 16), jnp.float32),
        ),
    )(x, W, S_a, S_b)


_sc_mesh = plsc.VectorSubcoreMesh(
    core_axis_name="c", subcore_axis_name="s", num_cores=NC, num_subcores=NS
)


@functools.partial(
    pl.kernel,
    out_type=jax.ShapeDtypeStruct((E // 2, 16), jnp.float32),
    mesh=_sc_mesh,
    scratch_types=[
        pltpu.VMEM((ROWS_PER_CHUNK, SUB), jnp.int32),
        pltpu.VMEM((ROWS_PER_CHUNK, SUB), jnp.int32),
        pltpu.VMEM((C, 16), jnp.float32),
        pltpu.VMEM((C, 16), jnp.float32),
        pltpu.VMEM((C // 2, 16), jnp.float32),
        pltpu.SemaphoreType.DMA,
        pltpu.SemaphoreType.DMA,
    ],
    compiler_params=pltpu.CompilerParams(use_tc_tiling_on_sc=False),
)
def _sc_edges(ta_hbm, tb_hbm, head_hbm, tail_hbm, out_hbm,
              hidx, tidx, ra, rb, ob, sem_a, sem_b):
    wid = lax.axis_index("s") * NC + lax.axis_index("c")
    row0 = wid * (PER_W // SUB)
    lane_lo = lax.iota(jnp.int32, 16) < 8

    def chunk_body(j, carry):
        rbase = pl.multiple_of(row0 + j * ROWS_PER_CHUNK, 8)
        pltpu.sync_copy(head_hbm.at[pl.ds(rbase, ROWS_PER_CHUNK)], hidx)
        pltpu.sync_copy(tail_hbm.at[pl.ds(rbase, ROWS_PER_CHUNK)], tidx)
        copies = []
        for k in range(ROWS_PER_CHUNK):
            copies.append(pltpu.async_copy(
                ta_hbm.at[hidx.at[k]], ra.at[pl.ds(k * SUB, SUB)], sem_a))
            copies.append(pltpu.async_copy(
                tb_hbm.at[tidx.at[k]], rb.at[pl.ds(k * SUB, SUB)], sem_b))
        for cp in copies:
            cp.wait()

        def pair_body(i, carry2):
            va = ra[2 * i, :] + rb[2 * i, :]
            vb = ra[2 * i + 1, :] + rb[2 * i + 1, :]
            sa = 1.0 / (1.0 + jnp.exp(va))
            sb = 1.0 / (1.0 + jnp.exp(vb))
            ob[i, :] = jnp.where(lane_lo, sa, sb)
            return carry2

        lax.fori_loop(0, C // 2, pair_body, 0, unroll=4)
        obase = pl.multiple_of((wid * PER_W + j * C) // 2, 8)
        pltpu.sync_copy(ob, out_hbm.at[pl.ds(obase, C // 2)])
        return carry

    lax.fori_loop(0, NCHUNK, chunk_body, 0)


def kernel(concept_hidden, head, tail, W, scoring_fn_source, scoring_fn_target):
    x = concept_hidden.astype(jnp.float32)
    hsel = (jnp.arange(IN) // OUT)[:, None] == jnp.arange(H)[None, :]
    onehot = hsel.astype(jnp.float32)
    s_src = -scoring_fn_source.reshape(IN)[:, None] * onehot
    s_tgt = -scoring_fn_target.reshape(IN)[:, None] * onehot
    S_a = jnp.concatenate([s_src, s_src], axis=1)
    S_b = jnp.concatenate([s_tgt, s_tgt], axis=1)

    table_a, table_b = _make_tables(x, W.astype(jnp.float32), S_a, S_b)

    head2d = head.astype(jnp.int32).reshape(E // SUB, SUB)
    tail2d = tail.astype(jnp.int32).reshape(E // SUB, SUB)
    out2 = _sc_edges(table_a, table_b, head2d, tail2d)
    return out2.reshape(E, H)
